# baseline (speedup 1.0000x reference)
import jax
import jax.numpy as jnp
from jax.experimental import pallas as pl


def _mlp_kernel(x_ref, w1_ref, b1_ref, w2_ref, b2_ref, o_ref):
    h = jnp.dot(x_ref[:], w1_ref[:], preferred_element_type=jnp.float32)
    h = jnp.maximum(h + b1_ref[:], 0.0)
    o_ref[:] = (
        jnp.dot(h, w2_ref[:], preferred_element_type=jnp.float32) + b2_ref[:]
    )


def kernel(x, edge_index, W1, b1, W2, b2):
    n, in_ch = x.shape
    hid = W1.shape[1]
    ncls = W2.shape[1]
    tile = 1000
    b1r = b1.reshape(1, hid)
    b2r = b2.reshape(1, ncls)
    return pl.pallas_call(
        _mlp_kernel,
        grid=(n // tile,),
        in_specs=[
            pl.BlockSpec((tile, in_ch), lambda i: (i, 0)),
            pl.BlockSpec((in_ch, hid), lambda i: (0, 0)),
            pl.BlockSpec((1, hid), lambda i: (0, 0)),
            pl.BlockSpec((hid, ncls), lambda i: (0, 0)),
            pl.BlockSpec((1, ncls), lambda i: (0, 0)),
        ],
        out_specs=pl.BlockSpec((tile, ncls), lambda i: (i, 0)),
        out_shape=jax.ShapeDtypeStruct((n, ncls), jnp.float32),
    )(x, W1, b1r, W2, b2r)


# speedup vs baseline: 1.1710x; 1.1710x over previous
import jax
import jax.numpy as jnp
from jax.experimental import pallas as pl
from jax.experimental.pallas import tpu as pltpu


def _mlp_kernel(x_ref, w1_ref, b1_ref, w2_ref, b2_ref, o_ref):
    h = jnp.dot(x_ref[:], w1_ref[:], preferred_element_type=jnp.float32)
    h = jnp.maximum(h + b1_ref[:], 0.0)
    o_ref[:] = (
        jnp.dot(h, w2_ref[:], preferred_element_type=jnp.float32) + b2_ref[:]
    )


def kernel(x, edge_index, W1, b1, W2, b2):
    n, in_ch = x.shape
    hid = W1.shape[1]
    ncls = W2.shape[1]
    tile = 2000
    b1r = b1.reshape(1, hid)
    b2r = b2.reshape(1, ncls)
    return pl.pallas_call(
        _mlp_kernel,
        grid=(n // tile,),
        in_specs=[
            pl.BlockSpec((tile, in_ch), lambda i: (i, 0)),
            pl.BlockSpec((in_ch, hid), lambda i: (0, 0)),
            pl.BlockSpec((1, hid), lambda i: (0, 0)),
            pl.BlockSpec((hid, ncls), lambda i: (0, 0)),
            pl.BlockSpec((1, ncls), lambda i: (0, 0)),
        ],
        out_specs=pl.BlockSpec((tile, ncls), lambda i: (i, 0)),
        out_shape=jax.ShapeDtypeStruct((n, ncls), jnp.float32),
        compiler_params=pltpu.CompilerParams(
            dimension_semantics=("parallel",),
        ),
    )(x, W1, b1r, W2, b2r)


# speedup vs baseline: 1.2806x; 1.0936x over previous
import jax
import jax.numpy as jnp
from jax.experimental import pallas as pl
from jax.experimental.pallas import tpu as pltpu


def _mlp_kernel(x_ref, w1_ref, b1_ref, w2_ref, b2_ref, o_ref):
    h = jnp.dot(x_ref[:], w1_ref[:], preferred_element_type=jnp.float32)
    h = jnp.maximum(h + b1_ref[:][None, :], 0.0)
    o_ref[:] = (
        jnp.dot(h, w2_ref[:], preferred_element_type=jnp.float32)
        + b2_ref[:][None, :]
    )


def kernel(x, edge_index, W1, b1, W2, b2):
    n = x.shape[0]
    ncls = W2.shape[1]
    return pl.pallas_call(
        _mlp_kernel,
        out_shape=jax.ShapeDtypeStruct((n, ncls), jnp.float32),
    )(x, W1, b1, W2, b2)
